# speedup vs baseline: 1.0457x; 1.0457x over previous
import functools

import jax
import jax.numpy as jnp
from jax import lax
from jax.experimental import pallas as pl
from jax.experimental.pallas import tpu as pltpu
from jax.experimental.pallas import tpu_sc as plsc

B, T, C = 1, 2048, 768
NH, NKV, HD = 12, 4, 64
E, K, H = 16, 1, 256
REP = NH // NKV
TB = 256
NTB = T // TB
BLK = 128
NB = T // BLK + E
TPAD = NB * BLK

SC_NC, SC_NS = 2, 16
NW = SC_NC * SC_NS
CHUNK = T // NW

_F32 = jnp.float32


def _pre_body(x_ref, n1_ref, wq_ref, wk_ref, wv_ref, cq_ref, sq_ref,
              ck_ref, sk_ref, rq_ref, rk_ref, q_ref, k_ref, v_ref):
    xb = x_ref[...]
    ms = jnp.mean(xb * xb, axis=-1, keepdims=True)
    hb = xb * lax.rsqrt(ms + 1e-6) * n1_ref[...]
    q = jnp.dot(hb, wq_ref[...], preferred_element_type=_F32)
    k = jnp.dot(hb, wk_ref[...], preferred_element_type=_F32)
    v = jnp.dot(hb, wv_ref[...], preferred_element_type=_F32)
    q = q * cq_ref[...] + jnp.dot(q, rq_ref[...], preferred_element_type=_F32) * sq_ref[...]
    k = k * ck_ref[...] + jnp.dot(k, rk_ref[...], preferred_element_type=_F32) * sk_ref[...]
    q_ref[...] = q
    k_ref[...] = k
    v_ref[...] = v


def _pre_call(x2d, n1, wqp, wkp, wv, cq, sq, ck, sk, rq, rk):
    return pl.pallas_call(
        _pre_body,
        grid=(NTB,),
        in_specs=[
            pl.BlockSpec((TB, C), lambda i: (i, 0)),
            pl.BlockSpec((1, C), lambda i: (0, 0)),
            pl.BlockSpec((C, NH * HD), lambda i: (0, 0)),
            pl.BlockSpec((C, NKV * HD), lambda i: (0, 0)),
            pl.BlockSpec((C, NKV * HD), lambda i: (0, 0)),
            pl.BlockSpec((TB, NH * HD), lambda i: (i, 0)),
            pl.BlockSpec((TB, NH * HD), lambda i: (i, 0)),
            pl.BlockSpec((TB, NKV * HD), lambda i: (i, 0)),
            pl.BlockSpec((TB, NKV * HD), lambda i: (i, 0)),
            pl.BlockSpec((NH * HD, NH * HD), lambda i: (0, 0)),
            pl.BlockSpec((NKV * HD, NKV * HD), lambda i: (0, 0)),
        ],
        out_specs=[
            pl.BlockSpec((TB, NH * HD), lambda i: (i, 0)),
            pl.BlockSpec((TB, NKV * HD), lambda i: (i, 0)),
            pl.BlockSpec((TB, NKV * HD), lambda i: (i, 0)),
        ],
        out_shape=[
            jax.ShapeDtypeStruct((T, NH * HD), _F32),
            jax.ShapeDtypeStruct((T, NKV * HD), _F32),
            jax.ShapeDtypeStruct((T, NKV * HD), _F32),
        ],
    )(x2d, n1, wqp, wkp, wv, cq, sq, ck, sk, rq, rk)


def _flash_body(q_ref, k_ref, v_ref, o_ref):
    qb = pl.program_id(1)
    q = q_ref[0] * _F32(1.0 / (HD ** 0.5))

    def step(kb, carry):
        acc, m, l = carry
        ks = k_ref[0, pl.ds(kb * TB, TB), :]
        vs = v_ref[0, pl.ds(kb * TB, TB), :]
        s = lax.dot_general(q, ks, (((1,), (1,)), ((), ())),
                            preferred_element_type=_F32)
        iq = lax.broadcasted_iota(jnp.int32, (TB, TB), 0) + qb * TB
        ik = lax.broadcasted_iota(jnp.int32, (TB, TB), 1) + kb * TB
        s = jnp.where(iq >= ik, s, _F32(-1e30))
        mn = jnp.maximum(m, jnp.max(s, axis=1, keepdims=True))
        p = jnp.exp(s - mn)
        alpha = jnp.exp(m - mn)
        l2 = l * alpha + jnp.sum(p, axis=1, keepdims=True)
        acc2 = acc * alpha + jnp.dot(p, vs, preferred_element_type=_F32)
        return acc2, mn, l2

    acc, _, l = lax.fori_loop(
        0, qb + 1, step,
        (jnp.zeros((TB, HD), _F32),
         jnp.full((TB, 1), -1e38, _F32),
         jnp.zeros((TB, 1), _F32)))
    o_ref[0] = acc / l


def _flash_call(q3, k3, v3):
    return pl.pallas_call(
        _flash_body,
        grid=(NH, NTB),
        in_specs=[
            pl.BlockSpec((1, TB, HD), lambda h, qb: (h, qb, 0)),
            pl.BlockSpec((1, T, HD), lambda h, qb: (h // REP, 0, 0)),
            pl.BlockSpec((1, T, HD), lambda h, qb: (h // REP, 0, 0)),
        ],
        out_specs=pl.BlockSpec((1, TB, HD), lambda h, qb: (h, qb, 0)),
        out_shape=jax.ShapeDtypeStruct((NH, T, HD), _F32),
    )(q3, k3, v3)


def _post_body(y_ref, x_ref, wo_ref, n2_ref, rw_ref, s1_ref, s2_ref, s3_ref,
               h2_ref, lg_ref, base_ref):
    x2 = x_ref[...] + jnp.dot(y_ref[...], wo_ref[...], preferred_element_type=_F32)
    ms = jnp.mean(x2 * x2, axis=-1, keepdims=True)
    h2 = x2 * lax.rsqrt(ms + 1e-6) * n2_ref[...]
    lg_ref[...] = jnp.dot(h2, rw_ref[...], preferred_element_type=_F32)
    g = jnp.dot(h2, s1_ref[...], preferred_element_type=_F32)
    u = jnp.dot(h2, s3_ref[...], preferred_element_type=_F32)
    sh = jnp.dot(jax.nn.silu(g) * u, s2_ref[...], preferred_element_type=_F32)
    h2_ref[...] = h2
    base_ref[...] = x2 + sh


def _post_call(y2d, x2d, wo, n2, rw, s1, s2, s3):
    return pl.pallas_call(
        _post_body,
        grid=(NTB,),
        in_specs=[
            pl.BlockSpec((TB, C), lambda i: (i, 0)),
            pl.BlockSpec((TB, C), lambda i: (i, 0)),
            pl.BlockSpec((C, C), lambda i: (0, 0)),
            pl.BlockSpec((1, C), lambda i: (0, 0)),
            pl.BlockSpec((C, E), lambda i: (0, 0)),
            pl.BlockSpec((C, H), lambda i: (0, 0)),
            pl.BlockSpec((H, C), lambda i: (0, 0)),
            pl.BlockSpec((C, H), lambda i: (0, 0)),
        ],
        out_specs=[
            pl.BlockSpec((TB, C), lambda i: (i, 0)),
            pl.BlockSpec((TB, E), lambda i: (i, 0)),
            pl.BlockSpec((TB, C), lambda i: (i, 0)),
        ],
        out_shape=[
            jax.ShapeDtypeStruct((T, C), _F32),
            jax.ShapeDtypeStruct((T, E), _F32),
            jax.ShapeDtypeStruct((T, C), _F32),
        ],
    )(y2d, x2d, wo, n2, rw, s1, s2, s3)


def _route_body(lg_ref, dest_ref, bexp_ref, act_ref):
    lg = lg_ref[...]
    rowmax = jnp.max(lg, axis=1, keepdims=True)
    ismax = (lg == rowmax).astype(_F32)
    ei = lax.broadcasted_iota(jnp.int32, (E, E), 0)
    ej = lax.broadcasted_iota(jnp.int32, (E, E), 1)
    minc = (ei <= ej).astype(_F32)
    cnt = jnp.dot(ismax, minc, preferred_element_type=_F32)
    oh = jnp.where((cnt == 1.0) & (ismax > 0.0), 1.0, 0.0)

    ri = lax.broadcasted_iota(jnp.int32, (TB, TB), 0)
    rj = lax.broadcasted_iota(jnp.int32, (TB, TB), 1)
    ltri = (rj < ri).astype(_F32)
    tot = jnp.zeros((1, E), _F32)
    chunks = []
    for c in range(NTB):
        ohc = oh[c * TB:(c + 1) * TB, :]
        chunks.append(jnp.dot(ltri, ohc, preferred_element_type=_F32) + tot)
        tot = tot + jnp.sum(ohc, axis=0, keepdims=True)
    ranks = jnp.concatenate(chunks, axis=0)

    counts = tot
    pc = jnp.ceil(counts / BLK) * BLK
    mstrict = (ei < ej).astype(_F32)
    offs = jnp.dot(pc, mstrict, preferred_element_type=_F32)

    dest = jnp.sum(oh * (offs + ranks), axis=1, keepdims=True)
    dest_ref[...] = dest.astype(jnp.int32)

    offb_col = jnp.sum((ei == ej).astype(_F32) * offs, axis=1, keepdims=True) / BLK
    bio = lax.broadcasted_iota(jnp.int32, (E, NB), 1).astype(_F32)
    cmp = (bio >= offb_col).astype(_F32)
    bexp_ref[...] = (jnp.sum(cmp, axis=0, keepdims=True) - 1.0).astype(jnp.int32)
    nact = jnp.sum(pc) / BLK
    bact = lax.broadcasted_iota(jnp.int32, (1, NB), 1).astype(_F32)
    act_ref[...] = (bact < nact).astype(jnp.int32)


def _route_call(logits):
    return pl.pallas_call(
        _route_body,
        grid=(1,),
        in_specs=[pl.BlockSpec((T, E), lambda i: (0, 0))],
        out_specs=[
            pl.BlockSpec((T, 1), lambda i: (0, 0)),
            pl.BlockSpec((1, NB), lambda i: (0, 0)),
            pl.BlockSpec((1, NB), lambda i: (0, 0)),
        ],
        out_shape=[
            jax.ShapeDtypeStruct((T, 1), jnp.int32),
            jax.ShapeDtypeStruct((1, NB), jnp.int32),
            jax.ShapeDtypeStruct((1, NB), jnp.int32),
        ],
    )(logits)


def _sc_mesh():
    return plsc.VectorSubcoreMesh(core_axis_name="c", subcore_axis_name="s")


def _dispatch_sc(h2, dest):
    @functools.partial(
        pl.kernel,
        mesh=_sc_mesh(),
        out_type=jax.ShapeDtypeStruct((TPAD, C), _F32),
        scratch_types=[
            pltpu.VMEM((CHUNK,), jnp.int32),
            pltpu.VMEM((CHUNK, C), _F32),
            pltpu.SemaphoreType.DMA,
        ],
    )
    def scatter_kernel(h2_hbm, dest_hbm, out_hbm, idx_v, rows_v, sem):
        wid = lax.axis_index("s") * SC_NC + lax.axis_index("c")
        base = wid * CHUNK
        pltpu.sync_copy(dest_hbm.at[pl.ds(base, CHUNK)], idx_v)
        pltpu.sync_copy(h2_hbm.at[pl.ds(base, CHUNK)], rows_v)
        pltpu.async_copy(rows_v, out_hbm.at[idx_v], sem).wait()

    return scatter_kernel(h2, dest)


def _combine_sc(eo, dest):
    @functools.partial(
        pl.kernel,
        mesh=_sc_mesh(),
        out_type=jax.ShapeDtypeStruct((T, C), _F32),
        scratch_types=[
            pltpu.VMEM((CHUNK,), jnp.int32),
            pltpu.VMEM((CHUNK, C), _F32),
            pltpu.SemaphoreType.DMA,
        ],
    )
    def gather_kernel(eo_hbm, dest_hbm, out_hbm, idx_v, rows_v, sem):
        wid = lax.axis_index("s") * SC_NC + lax.axis_index("c")
        base = wid * CHUNK
        pltpu.sync_copy(dest_hbm.at[pl.ds(base, CHUNK)], idx_v)
        pltpu.async_copy(eo_hbm.at[idx_v], rows_v, sem).wait()
        pltpu.sync_copy(rows_v, out_hbm.at[pl.ds(base, CHUNK)])

    return gather_kernel(eo, dest)


def _gemm_body(bexp_ref, act_ref, h_ref, w1_ref, w3_ref, w2_ref, o_ref):
    b = pl.program_id(0)

    @pl.when(act_ref[b] == 1)
    def _():
        hb = h_ref[...]
        g = jnp.dot(hb, w1_ref[0], preferred_element_type=_F32)
        u = jnp.dot(hb, w3_ref[0], preferred_element_type=_F32)
        o_ref[...] = jnp.dot(jax.nn.silu(g) * u, w2_ref[0],
                             preferred_element_type=_F32)


def _gemm_call(bexp, act, sorted_h, ew1, ew3, ew2):
    grid_spec = pltpu.PrefetchScalarGridSpec(
        num_scalar_prefetch=2,
        grid=(NB,),
        in_specs=[
            pl.BlockSpec((BLK, C), lambda b, bexp, act: (b, 0)),
            pl.BlockSpec((1, C, H), lambda b, bexp, act: (bexp[b], 0, 0)),
            pl.BlockSpec((1, C, H), lambda b, bexp, act: (bexp[b], 0, 0)),
            pl.BlockSpec((1, H, C), lambda b, bexp, act: (bexp[b], 0, 0)),
        ],
        out_specs=pl.BlockSpec((BLK, C), lambda b, bexp, act: (b, 0)),
    )
    return pl.pallas_call(
        _gemm_body,
        grid_spec=grid_spec,
        out_shape=jax.ShapeDtypeStruct((TPAD, C), _F32),
    )(bexp, act, sorted_h, ew1, ew3, ew2)


def _add_body(a_ref, b_ref, o_ref):
    o_ref[...] = a_ref[...] + b_ref[...]


def _add_call(a, b):
    return pl.pallas_call(
        _add_body,
        grid=(NTB,),
        in_specs=[
            pl.BlockSpec((TB, C), lambda i: (i, 0)),
            pl.BlockSpec((TB, C), lambda i: (i, 0)),
        ],
        out_specs=pl.BlockSpec((TB, C), lambda i: (i, 0)),
        out_shape=jax.ShapeDtypeStruct((T, C), _F32),
    )(a, b)


def _swapmat(nheads):
    n = nheads * HD
    i = jnp.arange(n)[:, None]
    j = jnp.arange(n)[None, :]
    same_head = (i // HD) == (j // HD)
    swapped = (i % HD) == ((j % HD) + HD // 2) % HD
    return (same_head & swapped).astype(_F32)


def kernel(x, freqs_cis, norm1_w, wq, wk, wv, wo, norm2_w, router_w,
           shared_w1, shared_w2, shared_w3, exp_w1, exp_w2, exp_w3):
    x2d = x.reshape(T, C)
    wqp = wq.reshape(C, NH, HD // 2, 2).transpose(0, 1, 3, 2).reshape(C, NH * HD)
    wkp = wk.reshape(C, NKV, HD // 2, 2).transpose(0, 1, 3, 2).reshape(C, NKV * HD)
    cos = jnp.cos(freqs_cis)
    sin = jnp.sin(freqs_cis)
    cs = jnp.concatenate([cos, cos], axis=1)
    ss = jnp.concatenate([-sin, sin], axis=1)
    cq, sq = jnp.tile(cs, (1, NH)), jnp.tile(ss, (1, NH))
    ck, sk = jnp.tile(cs, (1, NKV)), jnp.tile(ss, (1, NKV))
    rq, rk = _swapmat(NH), _swapmat(NKV)

    q2, k2, v2 = _pre_call(x2d, norm1_w.reshape(1, C), wqp, wkp, wv,
                           cq, sq, ck, sk, rq, rk)
    q3 = q2.reshape(T, NH, HD).transpose(1, 0, 2)
    k3 = k2.reshape(T, NKV, HD).transpose(1, 0, 2)
    v3 = v2.reshape(T, NKV, HD).transpose(1, 0, 2)
    y3 = _flash_call(q3, k3, v3)
    y2d = y3.transpose(1, 0, 2).reshape(T, C)

    h2, logits, base = _post_call(y2d, x2d, wo, norm2_w.reshape(1, C),
                                  router_w, shared_w1, shared_w2, shared_w3)

    dest2d, bexp2d, act2d = _route_call(logits)
    dest = dest2d.reshape(T)
    bexp = bexp2d.reshape(NB)
    act = act2d.reshape(NB)

    sorted_h = _dispatch_sc(h2, dest)
    eo = _gemm_call(bexp, act, sorted_h, exp_w1, exp_w3, exp_w2)
    moe = _combine_sc(eo, dest)
    out = _add_call(base, moe)

    return out.reshape(B, T, C), logits.reshape(B, T, E)
